# baseline (speedup 1.0000x reference)
import functools

import jax
import jax.numpy as jnp
from jax.experimental import pallas as pl
from jax.experimental.pallas import tpu as pltpu

_ALPHA = 16.0


def _moe_lora_body(x_ref, bias_ref, baug_ref, aaug_ref, out_ref,
                   bs_ref, h_ref, *, ts, nt, s_total, e, r):
    phase = pl.program_id(1)
    s = pl.program_id(2)
    er = e * r

    @pl.when(phase == 0)
    def _phase0():
        x = x_ref[0]
        h_ref[pl.ds(s * ts, ts), :] = jax.lax.dot_general(
            x.astype(jnp.bfloat16), aaug_ref[...], (((1,), (1,)), ((), ())),
            preferred_element_type=jnp.float32).astype(jnp.bfloat16)

    @pl.when((phase == 1) & (s == 0))
    def _router():
        g = h_ref[:, er:er + e].astype(jnp.float32)
        logits = (jnp.sum(g, axis=0, keepdims=True) * (1.0 / s_total)
                  + bias_ref[0])
        iota = jax.lax.broadcasted_iota(jnp.int32, (1, e), 1)
        v0 = jnp.max(logits, keepdims=True)
        i0 = jnp.min(jnp.where(logits == v0, iota, e), keepdims=True)
        masked = jnp.where(iota == i0, -jnp.inf, logits)
        v1 = jnp.max(masked, keepdims=True)
        i1 = jnp.min(jnp.where(masked == v1, iota, e), keepdims=True)
        t = jnp.exp(v1 - v0)
        w0 = 1.0 / (1.0 + t)
        w1 = t / (1.0 + t)
        scaling = _ALPHA / r
        eidx = jax.lax.broadcasted_iota(jnp.int32, (er + e, 1), 0) // r
        wrep_col = (jnp.where(eidx == i0, w0 * scaling, 0.0)
                    + jnp.where(eidx == i1, w1 * scaling, 0.0))
        bs_ref[...] = (baug_ref[...] * wrep_col).astype(jnp.bfloat16)

    @pl.when(phase == 1)
    def _phase1():
        h = h_ref[pl.ds(s * ts, ts), :]
        out_ref[0] = jax.lax.dot_general(
            h, bs_ref[...], (((1,), (0,)), ((), ())),
            preferred_element_type=jnp.float32)


def kernel(hidden_states, task_id, mode_id, W_gate, task_bias, mode_bias, A, Bw):
    b, s_total, d = hidden_states.shape
    e, r, _ = A.shape
    ts = 1024
    nt = s_total // ts

    a_aug = jnp.concatenate(
        [A.reshape(e * r, d), W_gate], axis=0).astype(jnp.bfloat16)
    b_aug = jnp.concatenate(
        [Bw.transpose(0, 2, 1).reshape(e * r, d),
         jnp.zeros((e, d), jnp.float32)], axis=0)
    bias = (jnp.take(task_bias, task_id, axis=0)
            + jnp.take(mode_bias, mode_id, axis=0))
    bias_row = bias.reshape(b, 1, e)

    body = functools.partial(_moe_lora_body, ts=ts, nt=nt,
                             s_total=s_total, e=e, r=r)

    return pl.pallas_call(
        body,
        grid=(b, 2, nt),
        in_specs=[
            pl.BlockSpec((1, ts, d),
                         lambda bi, p, si: (bi, jnp.where(p == 0, si, nt - 1), 0)),
            pl.BlockSpec((1, 1, e), lambda bi, p, si: (bi, 0, 0)),
            pl.BlockSpec((e * r + e, d), lambda bi, p, si: (0, 0)),
            pl.BlockSpec((e * r + e, d), lambda bi, p, si: (0, 0)),
        ],
        out_specs=pl.BlockSpec((1, ts, d),
                               lambda bi, p, si: (bi, jnp.where(p == 1, si, 0), 0)),
        out_shape=jax.ShapeDtypeStruct((b, s_total, d), jnp.float32),
        scratch_shapes=[
            pltpu.VMEM((e * r + e, d), jnp.bfloat16),
            pltpu.VMEM((s_total, e * r + e), jnp.bfloat16),
        ],
    )(hidden_states, bias_row, b_aug, a_aug)


# speedup vs baseline: 1.1728x; 1.1728x over previous
import functools

import jax
import jax.numpy as jnp
from jax.experimental import pallas as pl
from jax.experimental.pallas import tpu as pltpu

_ALPHA = 16.0


def _moe_lora_body(x_ref, bias_ref, baug_ref, aaug_ref, out_ref,
                   bs_ref, h_ref, *, ts, nt, s_total, e, r, nb):
    bi = pl.program_id(0)
    si = pl.program_id(1)
    er = e * r
    parity = jax.lax.rem(bi, 2)

    @pl.when(bi < nb)
    def _a_work():
        x = x_ref[0]
        h_ref[parity, pl.ds(si * ts, ts), :] = jax.lax.dot_general(
            x.astype(jnp.bfloat16), aaug_ref[...], (((1,), (1,)), ((), ())),
            preferred_element_type=jnp.float32).astype(jnp.bfloat16)

    @pl.when((bi >= 1) & (si == 0))
    def _router():
        g = h_ref[1 - parity, :, er:er + e].astype(jnp.float32)
        logits = (jnp.sum(g, axis=0, keepdims=True) * (1.0 / s_total)
                  + bias_ref[0])
        iota = jax.lax.broadcasted_iota(jnp.int32, (1, e), 1)
        v0 = jnp.max(logits, keepdims=True)
        i0 = jnp.min(jnp.where(logits == v0, iota, e), keepdims=True)
        masked = jnp.where(iota == i0, -jnp.inf, logits)
        v1 = jnp.max(masked, keepdims=True)
        i1 = jnp.min(jnp.where(masked == v1, iota, e), keepdims=True)
        t = jnp.exp(v1 - v0)
        w0 = 1.0 / (1.0 + t)
        w1 = t / (1.0 + t)
        scaling = _ALPHA / r
        eidx = jax.lax.broadcasted_iota(jnp.int32, (er + e, 1), 0) // r
        wrep_col = (jnp.where(eidx == i0, w0 * scaling, 0.0)
                    + jnp.where(eidx == i1, w1 * scaling, 0.0))
        bs_ref[...] = (baug_ref[...] * wrep_col).astype(jnp.bfloat16)

    @pl.when(bi >= 1)
    def _b_work():
        h = h_ref[1 - parity, pl.ds(si * ts, ts), :]
        out_ref[0] = jax.lax.dot_general(
            h, bs_ref[...], (((1,), (0,)), ((), ())),
            preferred_element_type=jnp.float32)


def kernel(hidden_states, task_id, mode_id, W_gate, task_bias, mode_bias, A, Bw):
    b, s_total, d = hidden_states.shape
    e, r, _ = A.shape
    ts = 512
    nt = s_total // ts

    a_aug = jnp.concatenate(
        [A.reshape(e * r, d), W_gate], axis=0).astype(jnp.bfloat16)
    b_aug = jnp.concatenate(
        [Bw.transpose(0, 2, 1).reshape(e * r, d),
         jnp.zeros((e, d), jnp.float32)], axis=0)
    bias = (jnp.take(task_bias, task_id, axis=0)
            + jnp.take(mode_bias, mode_id, axis=0))
    bias_row = bias.reshape(b, 1, e)

    body = functools.partial(_moe_lora_body, ts=ts, nt=nt,
                             s_total=s_total, e=e, r=r, nb=b)

    return pl.pallas_call(
        body,
        grid=(b + 1, nt),
        in_specs=[
            pl.BlockSpec((1, ts, d),
                         lambda bi, si, _b=b, _nt=nt: (
                             jnp.minimum(bi, _b - 1),
                             jnp.where(bi < _b, si, _nt - 1), 0)),
            pl.BlockSpec((1, 1, e),
                         lambda bi, si: (jnp.maximum(bi - 1, 0), 0, 0)),
            pl.BlockSpec((e * r + e, d), lambda bi, si: (0, 0)),
            pl.BlockSpec((e * r + e, d), lambda bi, si: (0, 0)),
        ],
        out_specs=pl.BlockSpec((1, ts, d),
                               lambda bi, si: (jnp.maximum(bi - 1, 0),
                                               jnp.where(bi >= 1, si, 0), 0)),
        out_shape=jax.ShapeDtypeStruct((b, s_total, d), jnp.float32),
        scratch_shapes=[
            pltpu.VMEM((e * r + e, d), jnp.bfloat16),
            pltpu.VMEM((2, s_total, e * r + e), jnp.bfloat16),
        ],
    )(hidden_states, bias_row, b_aug, a_aug)


# speedup vs baseline: 1.2801x; 1.0915x over previous
import functools

import jax
import jax.numpy as jnp
from jax.experimental import pallas as pl
from jax.experimental.pallas import tpu as pltpu

_ALPHA = 16.0


def _moe_lora_body(x_ref, bias_ref, baug_ref, aaug_ref, out_ref,
                   bs_ref, h_ref, *, ts, nt, s_total, e, r, nb):
    bi = pl.program_id(0)
    si = pl.program_id(1)
    er = e * r
    parity = jax.lax.rem(bi, 2)

    @pl.when(bi < nb)
    def _a_work():
        x = x_ref[0]
        h_ref[parity, pl.ds(si * ts, ts), :] = jax.lax.dot_general(
            x.astype(jnp.bfloat16), aaug_ref[...], (((1,), (1,)), ((), ())),
            preferred_element_type=jnp.float32).astype(jnp.bfloat16)

    @pl.when((bi >= 1) & (si == 0))
    def _router():
        g = h_ref[1 - parity, :, er:er + e].astype(jnp.float32)
        logits = (jnp.sum(g, axis=0, keepdims=True) * (1.0 / s_total)
                  + bias_ref[0])
        iota = jax.lax.broadcasted_iota(jnp.int32, (1, e), 1)
        v0 = jnp.max(logits, keepdims=True)
        i0 = jnp.min(jnp.where(logits == v0, iota, e), keepdims=True)
        masked = jnp.where(iota == i0, -jnp.inf, logits)
        v1 = jnp.max(masked, keepdims=True)
        i1 = jnp.min(jnp.where(masked == v1, iota, e), keepdims=True)
        t = jnp.exp(v1 - v0)
        w0 = 1.0 / (1.0 + t)
        w1 = t / (1.0 + t)
        scaling = _ALPHA / r
        eidx = jax.lax.broadcasted_iota(jnp.int32, (er + e, 1), 0) // r
        wrep_col = (jnp.where(eidx == i0, w0 * scaling, 0.0)
                    + jnp.where(eidx == i1, w1 * scaling, 0.0))
        bs_ref[...] = (baug_ref[...] * wrep_col).astype(jnp.bfloat16)

    @pl.when(bi >= 1)
    def _b_work():
        h = h_ref[1 - parity, pl.ds(si * ts, ts), :]
        out_ref[0] = jax.lax.dot_general(
            h, bs_ref[...], (((1,), (0,)), ((), ())),
            preferred_element_type=jnp.float32)


def kernel(hidden_states, task_id, mode_id, W_gate, task_bias, mode_bias, A, Bw):
    b, s_total, d = hidden_states.shape
    e, r, _ = A.shape
    ts = 1024
    nt = s_total // ts

    a_aug = jnp.concatenate(
        [A.reshape(e * r, d), W_gate], axis=0).astype(jnp.bfloat16)
    b_aug = jnp.concatenate(
        [Bw.transpose(0, 2, 1).reshape(e * r, d),
         jnp.zeros((e, d), jnp.float32)], axis=0)
    bias = (jnp.take(task_bias, task_id, axis=0)
            + jnp.take(mode_bias, mode_id, axis=0))
    bias_row = bias.reshape(b, 1, e)

    body = functools.partial(_moe_lora_body, ts=ts, nt=nt,
                             s_total=s_total, e=e, r=r, nb=b)

    return pl.pallas_call(
        body,
        grid=(b + 1, nt),
        in_specs=[
            pl.BlockSpec((1, ts, d),
                         lambda bi, si, _b=b, _nt=nt: (
                             jnp.minimum(bi, _b - 1),
                             jnp.where(bi < _b, si, _nt - 1), 0)),
            pl.BlockSpec((1, 1, e),
                         lambda bi, si: (jnp.maximum(bi - 1, 0), 0, 0)),
            pl.BlockSpec((e * r + e, d), lambda bi, si: (0, 0)),
            pl.BlockSpec((e * r + e, d), lambda bi, si: (0, 0)),
        ],
        out_specs=pl.BlockSpec((1, ts, d),
                               lambda bi, si: (jnp.maximum(bi - 1, 0),
                                               jnp.where(bi >= 1, si, 0), 0)),
        out_shape=jax.ShapeDtypeStruct((b, s_total, d), jnp.float32),
        scratch_shapes=[
            pltpu.VMEM((e * r + e, d), jnp.bfloat16),
            pltpu.VMEM((2, s_total, e * r + e), jnp.bfloat16),
        ],
    )(hidden_states, bias_row, b_aug, a_aug)
